# Initial kernel scaffold; baseline (speedup 1.0000x reference)
import jax
import jax.numpy as jnp
from jax.experimental import pallas as pl


def kernel(node_2D_repr, positions, pos_noise, t_graph, batch, edge_index, extended_edge_attr, anneal_power, W_node, b_node, W_e2d, b_e2d, emb_table, W_fourier, W_coffmlp, b_coffmlp, W_p1, b_p1, W_p2, b_p2, W_s1, b_s1, W_s2, b_s2):
    raise NotImplementedError("write your pallas kernel here")


# speedup vs baseline: 3.1296x; 3.1296x over previous
import functools

import jax
import jax.numpy as jnp
import numpy as np
from jax import lax
from jax.experimental import pallas as pl
from jax.experimental.pallas import tpu as pltpu
from jax.experimental.pallas import tpu_sc as plsc

EPS = 1e-6
N = 10000
E = 160000
G = 100
H = 128
SIGMA_MIN = 0.01
SIGMA_MAX = 2.0
LOG_RATIO = float(np.log(SIGMA_MAX / SIGMA_MIN))
LOG_SMIN = float(np.log(SIGMA_MIN))
TWO_PI = float(2.0 * np.pi)

BLK_N = 2000
BLK_E = 1280
SC_WIN = 128

_SC_PARAMS = pltpu.CompilerParams(needs_layout_passes=False)


def _s1_body(node_ref, pos4_ref, noise4_ref, batch_ref, tg_ref, ap_ref,
             we2d_a_ref, we2d_b_ref, wnode_ref, bnode_ref, ws1a_ref, ws1b_ref,
             rt_ref, ct_ref, posp_ref):
    node = node_ref[...]
    na = jnp.dot(node, wnode_ref[...], preferred_element_type=jnp.float32)
    na = na + bnode_ref[...]
    rt_ref[:, 0:H] = jnp.dot(node, we2d_a_ref[...],
                             preferred_element_type=jnp.float32)
    rt_ref[:, H:2 * H] = jnp.dot(na, ws1a_ref[...],
                                 preferred_element_type=jnp.float32)
    ct_ref[:, 0:H] = jnp.dot(node, we2d_b_ref[...],
                             preferred_element_type=jnp.float32)
    ct_ref[:, H:2 * H] = jnp.dot(na, ws1b_ref[...],
                                 preferred_element_type=jnp.float32)

    batch = batch_ref[...]
    onehot = (batch == lax.broadcasted_iota(jnp.int32, (BLK_N, 128), 1))
    t_pos = jnp.dot(onehot.astype(jnp.float32), tg_ref[...],
                    preferred_element_type=jnp.float32)
    log_std = LOG_SMIN + t_pos * LOG_RATIO
    std = jnp.exp(log_std)
    annealed = jnp.exp(ap_ref[...] * log_std)
    pp = pos4_ref[...] + std * noise4_ref[...]
    lane3 = (lax.broadcasted_iota(jnp.int32, (BLK_N, 4), 1) == 3)
    posp_ref[...] = pp + jnp.where(lane3, annealed, 0.0)


def _stage1(node, pos4, noise4, batch2d, tgpad, ap, we2d_a, we2d_b, wnode,
            bnode, ws1a, ws1b):
    grid = N // BLK_N
    full = lambda shape: pl.BlockSpec(shape, lambda i: (0, 0))
    return pl.pallas_call(
        _s1_body,
        grid=(grid,),
        in_specs=[
            pl.BlockSpec((BLK_N, H), lambda i: (i, 0)),
            pl.BlockSpec((BLK_N, 4), lambda i: (i, 0)),
            pl.BlockSpec((BLK_N, 4), lambda i: (i, 0)),
            pl.BlockSpec((BLK_N, 1), lambda i: (i, 0)),
            full((128, 1)),
            full((1, 1)),
            full((H, H)), full((H, H)), full((H, H)), full((1, H)),
            full((H, H)), full((H, H)),
        ],
        out_specs=[
            pl.BlockSpec((BLK_N, 2 * H), lambda i: (i, 0)),
            pl.BlockSpec((BLK_N, 2 * H), lambda i: (i, 0)),
            pl.BlockSpec((BLK_N, 4), lambda i: (i, 0)),
        ],
        out_shape=[
            jax.ShapeDtypeStruct((N, 2 * H), jnp.float32),
            jax.ShapeDtypeStruct((N, 2 * H), jnp.float32),
            jax.ShapeDtypeStruct((N, 4), jnp.float32),
        ],
    )(node, pos4, noise4, batch2d, tgpad, ap, we2d_a, we2d_b, wnode, bnode,
      ws1a, ws1b)


def _stage2(rt, ct, posp_flat, row2d, col2d):
    mesh = plsc.VectorSubcoreMesh(core_axis_name="c", subcore_axis_name="s")

    @functools.partial(
        pl.kernel,
        out_type=[
            jax.ShapeDtypeStruct((E, 2 * H), jnp.float32),
            jax.ShapeDtypeStruct((E, 2 * H), jnp.float32),
            jax.ShapeDtypeStruct((4, E), jnp.float32),
            jax.ShapeDtypeStruct((4, E), jnp.float32),
        ],
        mesh=mesh,
        scratch_types=[pltpu.VMEM((4 * N,), jnp.float32)],
        compiler_params=_SC_PARAMS,
    )
    def sc_gather(rt_hbm, ct_hbm, pos_hbm, row_hbm, col_hbm,
                  rtg_hbm, ctg_hbm, pit_hbm, pjt_hbm, table_v):
        pltpu.sync_copy(pos_hbm, table_v)
        zeros16 = jnp.zeros((16,), jnp.float32)

        def make_body(tab_hbm):
            def body(idx_v, g_v, pt_v):
                pltpu.sync_copy(tab_hbm.at[idx_v.at[0]], g_v)
                for s in range(SC_WIN // 16):
                    i16 = idx_v[0, pl.ds(s * 16, 16)]
                    b4 = i16 * 4
                    for c in range(3):
                        vals = plsc.load_gather(table_v, [b4 + c])
                        pt_v[c, pl.ds(s * 16, 16)] = vals
                    pt_v[3, pl.ds(s * 16, 16)] = zeros16
            return body

        for tab, idx2d, g_out, pt_out in (
            (rt_hbm, row_hbm, rtg_hbm, pit_hbm),
            (ct_hbm, col_hbm, ctg_hbm, pjt_hbm),
        ):
            pltpu.emit_pipeline(
                make_body(tab),
                grid=(E // SC_WIN,),
                in_specs=[pl.BlockSpec((1, SC_WIN), lambda i: (0, i))],
                out_specs=[
                    pl.BlockSpec((SC_WIN, 2 * H), lambda i: (i, 0)),
                    pl.BlockSpec((4, SC_WIN), lambda i: (0, i)),
                ],
                core_axis_name=("c", "s"),
                dimension_semantics=(pltpu.PARALLEL,),
            )(idx2d, g_out, pt_out)

    return sc_gather(rt, ct, posp_flat, row2d, col2d)


def _silu(x):
    return x / (1.0 + jnp.exp(-x))


def _stage3(rtg, ctg, pit, pjt, attr2d, embpad, wf_row, wcs0, wcc0, wcs2,
            wcc2, bcoff, wp1ang, wp1a, wp1b, bp1, wp2, bp2, be2d, ws1c, bs1,
            ws2p, bs2p):
    def body(rtg_ref, ctg_ref, pi_ref, pj_ref, attr_ref, embpad_ref, wf_ref,
             wcs0_ref, wcc0_ref, wcs2_ref, wcc2_ref, bcoff_ref,
             wp1ang_ref, wp1a_ref, wp1b_ref, bp1_ref, wp2_ref, bp2_ref,
             be2d_ref, ws1c_ref, bs1_ref, ws2p_ref, bs2p_ref, bas_ref):
        f32 = jnp.float32
        ef = rtg_ref[...] + ctg_ref[...]
        pit = pi_ref[...]
        pjt = pj_ref[...]
        xi, yi, zi = pit[0:1], pit[1:2], pit[2:3]
        xj, yj, zj = pjt[0:1], pjt[1:2], pjt[2:3]
        dx, dy, dz = xi - xj, yi - yj, zi - zj
        radial = dx * dx + dy * dy + dz * dz
        inv_n = 1.0 / (jnp.sqrt(radial) + EPS)
        cdx, cdy, cdz = dx * inv_n, dy * inv_n, dz * inv_n
        cx = yi * zj - zi * yj
        cy = zi * xj - xi * zj
        cz = xi * yj - yi * xj
        inv_c = 1.0 / (jnp.sqrt(cx * cx + cy * cy + cz * cz) + EPS)
        ccx, ccy, ccz = cx * inv_c, cy * inv_c, cz * inv_c
        vx = cdy * ccz - cdz * ccy
        vy = cdz * ccx - cdx * ccz
        vz = cdx * ccy - cdy * ccx
        ci0 = cdx * xi + cdy * yi + cdz * zi
        ci1 = jnp.abs(ccx * xi + ccy * yi + ccz * zi)
        ci2 = vx * xi + vy * yi + vz * zi
        cj0 = cdx * xj + cdy * yj + cdz * zj
        cj1 = jnp.abs(ccx * xj + ccy * yj + ccz * zj)
        cj2 = vx * xj + vy * yj + vz * zj
        mul = ci0 * cj0 + ci1 * cj1 + ci2 * cj2
        ni = jnp.sqrt(ci0 * ci0 + ci1 * ci1 + ci2 * ci2) + EPS
        nj = jnp.sqrt(cj0 * cj0 + cj1 * cj1 + cj2 * cj2) + EPS
        pcos = mul / ni / nj
        psin = jnp.sqrt(jnp.clip(1.0 - pcos * pcos, EPS, 1.0))
        zrow = jnp.zeros_like(ci0)
        c8 = jnp.concatenate(
            [ci0, ci2, cj0, cj2, psin, pcos, zrow, zrow], axis=0).T

        wf = wf_ref[...]
        arg_i0 = c8[:, 0:1] * wf
        arg_i2 = c8[:, 1:2] * wf
        arg_j0 = c8[:, 2:3] * wf
        arg_j2 = c8[:, 3:4] * wf
        dot = lambda a, b: jnp.dot(a, b, preferred_element_type=f32)
        embed_i = (dot(jnp.sin(arg_i0), wcs0_ref[...])
                   + dot(jnp.cos(arg_i0), wcc0_ref[...])
                   + dot(jnp.sin(arg_i2), wcs2_ref[...])
                   + dot(jnp.cos(arg_i2), wcc2_ref[...]) + bcoff_ref[...])
        embed_j = (dot(jnp.sin(arg_j0), wcs0_ref[...])
                   + dot(jnp.cos(arg_j0), wcc0_ref[...])
                   + dot(jnp.sin(arg_j2), wcs2_ref[...])
                   + dot(jnp.cos(arg_j2), wcc2_ref[...]) + bcoff_ref[...])

        h = _silu(dot(c8, wp1ang_ref[...]) + dot(embed_i, wp1a_ref[...])
                  + dot(embed_j, wp1b_ref[...]) + bp1_ref[...])
        e3d = dot(h, wp2_ref[...]) + bp2_ref[...]

        attr = attr_ref[...]
        att_oh = (attr == lax.broadcasted_iota(jnp.int32, (BLK_E, 128), 1))
        emb = dot(att_oh.astype(f32), embpad_ref[...])
        edge_attr = ef[:, 0:H] + be2d_ref[...] + emb + e3d

        hc = _silu(ef[:, H:2 * H] + dot(edge_attr, ws1c_ref[...])
                   + bs1_ref[...])
        coffs = dot(hc, ws2p_ref[...]) + bs2p_ref[...]
        s8 = coffs.T
        s0, s1, s2 = s8[0:1], s8[1:2], s8[2:3]
        bx = s0 * cdx + s1 * ccx + s2 * vx
        by = s0 * cdy + s1 * ccy + s2 * vy
        bz = s0 * cdz + s1 * ccz + s2 * vz
        bas_ref[...] = jnp.concatenate([bx, by, bz, zrow], axis=0).T

    grid = E // BLK_E
    full = lambda r, c: pl.BlockSpec((r, c), lambda i: (0, 0))
    return pl.pallas_call(
        body,
        grid=(grid,),
        in_specs=[
            pl.BlockSpec((BLK_E, 2 * H), lambda i: (i, 0)),
            pl.BlockSpec((BLK_E, 2 * H), lambda i: (i, 0)),
            pl.BlockSpec((4, BLK_E), lambda i: (0, i)),
            pl.BlockSpec((4, BLK_E), lambda i: (0, i)),
            pl.BlockSpec((BLK_E, 1), lambda i: (i, 0)),
            full(128, H), full(1, H),
            full(H, H), full(H, H), full(H, H), full(H, H), full(1, H),
            full(8, H), full(H, H), full(H, H), full(1, H),
            full(H, H), full(1, H), full(1, H),
            full(H, H), full(1, H), full(H, 8), full(1, 8),
        ],
        out_specs=pl.BlockSpec((BLK_E, 4), lambda i: (i, 0)),
        out_shape=jax.ShapeDtypeStruct((E, 4), jnp.float32),
    )(rtg, ctg, pit, pjt, attr2d, embpad, wf_row, wcs0, wcc0, wcs2, wcc2,
      bcoff, wp1ang, wp1a, wp1b, bp1, wp2, bp2, be2d, ws1c, bs1, ws2p, bs2p)


NPAD = 10240
ACC = 4 * NPAD
SLICE = ACC // 16


def _stage4(bas, rowwin, zeros_acc):
    mesh = plsc.VectorSubcoreMesh(core_axis_name="c", subcore_axis_name="s")
    n_win = E // SC_WIN

    @functools.partial(
        pl.kernel,
        out_type=jax.ShapeDtypeStruct((2, ACC), jnp.float32),
        mesh=mesh,
        scratch_types=[
            pltpu.VMEM((ACC,), jnp.float32),
            pltpu.VMEM((SC_WIN, 4), jnp.float32),
            pltpu.VMEM((SC_WIN,), jnp.int32),
            pltpu.VMEM((SLICE,), jnp.float32),
            pltpu.VMEM((SLICE,), jnp.float32),
            pltpu.VMEM_SHARED((16, ACC), jnp.float32),
        ],
        compiler_params=_SC_PARAMS,
    )
    def sc_scatter(bas_hbm, row_hbm, zeros_hbm, out_hbm,
                   acc_v, basw_v, idxw_v, red_v, tmp_v, shared):
        cid = lax.axis_index("c")
        sid = lax.axis_index("s")
        wid = sid * 2 + cid
        pltpu.sync_copy(zeros_hbm, acc_v)
        base = 39 * wid + jnp.minimum(wid, 2)
        count = 39 + (wid < 2).astype(jnp.int32)
        iota16 = lax.iota(jnp.int32, 16)

        @pl.loop(base, base + count)
        def _(w):
            pltpu.sync_copy(bas_hbm.at[pl.ds(w * SC_WIN, SC_WIN)], basw_v)
            pltpu.sync_copy(row_hbm.at[pl.ds(w * SC_WIN, SC_WIN)], idxw_v)
            for s in range(SC_WIN // 16):
                i16 = idxw_v[pl.ds(s * 16, 16)]
                b4 = i16 * 4
                src_r = iota16 + (s * 16)
                for c in range(3):
                    vals = plsc.load_gather(
                        basw_v, [src_r, jnp.full((16,), c, jnp.int32)])
                    plsc.addupdate_scatter(acc_v, [b4 + c], vals)

        pltpu.sync_copy(acc_v, shared.at[sid])
        plsc.subcore_barrier()
        off = sid * SLICE
        pltpu.sync_copy(shared.at[0, pl.ds(off, SLICE)], red_v)
        for t in range(1, 16):
            pltpu.sync_copy(shared.at[t, pl.ds(off, SLICE)], tmp_v)

            @pl.loop(0, SLICE // 16)
            def _(k):
                sl = pl.ds(k * 16, 16)
                red_v[sl] = red_v[sl] + tmp_v[sl]

        pltpu.sync_copy(red_v, out_hbm.at[cid, pl.ds(off, SLICE)])

    return sc_scatter(bas, rowwin, zeros_acc)


def _stage5(scores2, noise4, posp, batch2d):
    blk = 2048
    grid = NPAD // blk

    def body(s0_ref, s1_ref, noise_ref, posp_ref, batch_ref, out_ref,
             sums_ref, cnts_ref):
        i = pl.program_id(0)

        @pl.when(i == 0)
        def _():
            sums_ref[...] = jnp.zeros_like(sums_ref)
            cnts_ref[...] = jnp.zeros_like(cnts_ref)

        diff = s0_ref[...] + s1_ref[...] - noise_ref[...]
        ann = posp_ref[:, 3:4]
        sq = diff * diff * ann
        rowsum = jnp.sum(sq, axis=1, keepdims=True)
        onehot = (batch_ref[...] ==
                  lax.broadcasted_iota(jnp.int32, (blk, 128), 1))
        ohf = onehot.astype(jnp.float32)
        sums_ref[...] += jnp.sum(ohf * rowsum, axis=0, keepdims=True)
        cnts_ref[...] += jnp.sum(ohf, axis=0, keepdims=True)

        @pl.when(i == grid - 1)
        def _():
            lg = sums_ref[...] / jnp.maximum(cnts_ref[...], 1.0)
            out_ref[...] = jnp.sum(lg, axis=1, keepdims=True) / float(G)

    return pl.pallas_call(
        body,
        grid=(grid,),
        in_specs=[
            pl.BlockSpec((blk, 4), lambda i: (i, 0)),
            pl.BlockSpec((blk, 4), lambda i: (i, 0)),
            pl.BlockSpec((blk, 4), lambda i: (i, 0)),
            pl.BlockSpec((blk, 4), lambda i: (i, 0)),
            pl.BlockSpec((blk, 1), lambda i: (i, 0)),
        ],
        out_specs=pl.BlockSpec((1, 1), lambda i: (0, 0)),
        out_shape=jax.ShapeDtypeStruct((1, 1), jnp.float32),
        scratch_shapes=[
            pltpu.VMEM((1, 128), jnp.float32),
            pltpu.VMEM((1, 128), jnp.float32),
        ],
    )(scores2[0], scores2[1], noise4, posp, batch2d)


def kernel(node_2D_repr, positions, pos_noise, t_graph, batch, edge_index,
           extended_edge_attr, anneal_power, W_node, b_node, W_e2d, b_e2d,
           emb_table, W_fourier, W_coffmlp, b_coffmlp, W_p1, b_p1, W_p2,
           b_p2, W_s1, b_s1, W_s2, b_s2):
    f32 = jnp.float32
    pad4 = lambda x: jnp.pad(x, ((0, 0), (0, 1)))
    pos4 = pad4(positions)
    noise4 = pad4(pos_noise)
    batch2d = batch.reshape(N, 1)
    tgpad = jnp.pad(t_graph, (0, 128 - G)).reshape(128, 1)
    ap = jnp.asarray(anneal_power, f32).reshape(1, 1)

    rt, ct, posp = _stage1(
        node_2D_repr, pos4, noise4, batch2d, tgpad, ap,
        W_e2d[0:H], W_e2d[H:2 * H], W_node, b_node.reshape(1, H),
        W_s1[0:H], W_s1[H:2 * H])

    row2d = edge_index[0].reshape(1, E)
    col2d = edge_index[1].reshape(1, E)
    rtg, ctg, pit, pjt = _stage2(rt, ct, posp.reshape(4 * N), row2d, col2d)

    attr2d = extended_edge_attr.reshape(E, 1)
    embpad = jnp.pad(emb_table, ((0, 128 - emb_table.shape[0]), (0, 0)))
    wf_row = (W_fourier * TWO_PI).reshape(1, H)
    wp1ang = jnp.zeros((8, H), f32).at[4].set(W_p1[0]).at[5].set(W_p1[1])
    ws2p = jnp.pad(W_s2, ((0, 0), (0, 5)))
    bs2p = jnp.pad(b_s2, (0, 5)).reshape(1, 8)

    bas = _stage3(
        rtg, ctg, pit, pjt, attr2d, embpad, wf_row,
        W_coffmlp[0:H], W_coffmlp[H:2 * H], W_coffmlp[2 * H:3 * H],
        W_coffmlp[3 * H:4 * H], b_coffmlp.reshape(1, H),
        wp1ang, W_p1[2:2 + H], W_p1[2 + H:2 + 2 * H], b_p1.reshape(1, H),
        W_p2, b_p2.reshape(1, H), b_e2d.reshape(1, H),
        W_s1[2 * H:3 * H], b_s1.reshape(1, H), ws2p, bs2p)

    zeros_acc = jnp.zeros((ACC,), f32)
    scores2 = _stage4(bas, edge_index[0], zeros_acc)
    scores2 = scores2.reshape(2, NPAD, 4)

    padn = NPAD - N
    noise4p = jnp.pad(noise4, ((0, padn), (0, 0)))
    posp_p = jnp.pad(posp, ((0, padn), (0, 0)))
    batch2d_p = jnp.pad(batch2d, ((0, padn), (0, 0)), constant_values=127)

    out = _stage5(scores2, noise4p, posp_p, batch2d_p)
    return out[0, 0]
